# Initial kernel scaffold; baseline (speedup 1.0000x reference)
import jax
import jax.numpy as jnp
from jax.experimental import pallas as pl


def kernel(x, edge_index, edge_attr, W1, b1, W2, b2, W3, b3, W4, b4):
    raise NotImplementedError("write your pallas kernel here")


# speedup vs baseline: 2.1538x; 2.1538x over previous
import jax
import jax.numpy as jnp
from jax import lax
from jax.experimental import pallas as pl
from jax.experimental.pallas import tpu as pltpu
from jax.experimental.pallas import tpu_sc as plsc

N = 10000
E = 320000
D = 128
DE = 16
H = 64
OUT = 128
PW = 144

NC, NS = 2, 16
NW = NC * NS
EPW = E // NW
CH = 80
NCH = EPW // CH
RPT = N // NS
RZ = 125
import functools


@functools.cache
def _mesh():
    return plsc.VectorSubcoreMesh(core_axis_name="c", subcore_axis_name="s",
                                  num_cores=NC, num_subcores=NS)

_NB = 1000


def _proj_body(x_ref, wa_ref, wb_ref, oa_ref, ob_ref):
    xb = x_ref[...]
    oa_ref[...] = jnp.dot(xb, wa_ref[...], preferred_element_type=jnp.float32)
    ob_ref[...] = jnp.dot(xb, wb_ref[...], preferred_element_type=jnp.float32)


def _proj(x, wat, wbt):
    return pl.pallas_call(
        _proj_body,
        grid=(N // _NB,),
        in_specs=[pl.BlockSpec((_NB, D), lambda i: (i, 0)),
                  pl.BlockSpec((D, H), lambda i: (0, 0)),
                  pl.BlockSpec((D, H), lambda i: (0, 0))],
        out_specs=[pl.BlockSpec((_NB, H), lambda i: (i, 0)),
                   pl.BlockSpec((_NB, H), lambda i: (i, 0))],
        out_shape=[jax.ShapeDtypeStruct((N, H), jnp.float32),
                   jax.ShapeDtypeStruct((N, H), jnp.float32)],
    )(x, wat, wbt)


def _gather_body(xa, xb, src, dst, ga, gb, sidx, didx, abuf, bbuf, sema, semb):
    wid = lax.axis_index("s") * NC + lax.axis_index("c")
    base = wid * EPW

    def step(i, carry):
        off = base + i * CH
        pltpu.sync_copy(src.at[pl.ds(off, CH)], sidx)
        pltpu.sync_copy(dst.at[pl.ds(off, CH)], didx)
        ca = pltpu.async_copy(xa.at[sidx], abuf, sema)
        cb = pltpu.async_copy(xb.at[didx], bbuf, semb)
        ca.wait()
        cb.wait()
        pltpu.sync_copy(abuf, ga.at[pl.ds(off, CH)])
        pltpu.sync_copy(bbuf, gb.at[pl.ds(off, CH)])
        return carry

    lax.fori_loop(0, NCH, step, 0)


@functools.cache
def _gather():
    return pl.kernel(
        _gather_body,
        out_type=(jax.ShapeDtypeStruct((E, H), jnp.float32),
                  jax.ShapeDtypeStruct((E, H), jnp.float32)),
        mesh=_mesh(),
        scratch_types=[pltpu.VMEM((CH,), jnp.int32),
                       pltpu.VMEM((CH,), jnp.int32),
                       pltpu.VMEM((CH, H), jnp.float32),
                       pltpu.VMEM((CH, H), jnp.float32),
                       pltpu.SemaphoreType.DMA,
                       pltpu.SemaphoreType.DMA],
        compiler_params=pltpu.CompilerParams(use_tc_tiling_on_sc=False),
    )

_BE = 2000


def _mlp_body(ga, gb, ea, w1ct, b1, w2t, b2, w3t, b3, w4t, b4, out):
    h = (ga[...] + gb[...]
         + jnp.dot(ea[...], w1ct[...], preferred_element_type=jnp.float32)
         + b1[...])
    h = jnp.maximum(h, 0.0)
    h = jnp.maximum(
        jnp.dot(h, w2t[...], preferred_element_type=jnp.float32) + b2[...], 0.0)
    h = jnp.maximum(
        jnp.dot(h, w3t[...], preferred_element_type=jnp.float32) + b3[...], 0.0)
    m = jnp.dot(h, w4t[...], preferred_element_type=jnp.float32) + b4[...]
    out[...] = jnp.concatenate(
        [m, jnp.ones((_BE, PW - OUT), jnp.float32)], axis=1)


def _edge_mlp(ga, gb, ea, w1ct, b1, w2t, b2, w3t, b3, w4t, b4):
    zero = lambda i: (0, 0)
    return pl.pallas_call(
        _mlp_body,
        grid=(E // _BE,),
        in_specs=[pl.BlockSpec((_BE, H), lambda i: (i, 0)),
                  pl.BlockSpec((_BE, H), lambda i: (i, 0)),
                  pl.BlockSpec((_BE, DE), lambda i: (i, 0)),
                  pl.BlockSpec((DE, H), zero),
                  pl.BlockSpec((1, H), zero),
                  pl.BlockSpec((H, H), zero),
                  pl.BlockSpec((1, H), zero),
                  pl.BlockSpec((H, H), zero),
                  pl.BlockSpec((1, H), zero),
                  pl.BlockSpec((H, OUT), zero),
                  pl.BlockSpec((1, OUT), zero)],
        out_specs=pl.BlockSpec((_BE, PW), lambda i: (i, 0)),
        out_shape=jax.ShapeDtypeStruct((E, PW), jnp.float32),
    )(ga, gb, ea, w1ct, b1, w2t, b2, w3t, b3, w4t, b4)


def _scatter_body(msg, dst, zrows, out, didx, mbuf, zbuf, sem, acc):
    cid = lax.axis_index("c")
    sid = lax.axis_index("s")
    pltpu.sync_copy(zrows, zbuf)
    for k in range(RPT // RZ):
        pltpu.sync_copy(zbuf, acc.at[pl.ds(sid * RPT + k * RZ, RZ)])
    plsc.subcore_barrier()

    wid = sid * NC + cid
    base = wid * EPW

    def step(i, carry):
        off = base + i * CH
        pltpu.sync_copy(dst.at[pl.ds(off, CH)], didx)
        pltpu.sync_copy(msg.at[pl.ds(off, CH)], mbuf)
        pltpu.sync_copy(mbuf, acc.at[didx], add=True)
        return carry

    lax.fori_loop(0, NCH, step, 0)
    plsc.subcore_barrier()
    for k in range(RPT // RZ):
        r0 = sid * RPT + k * RZ
        pltpu.sync_copy(acc.at[pl.ds(r0, RZ)], zbuf)
        pltpu.sync_copy(zbuf, out.at[cid, pl.ds(r0, RZ)])


@functools.cache
def _scatter():
    return pl.kernel(
        _scatter_body,
        out_type=jax.ShapeDtypeStruct((NC, N, PW), jnp.float32),
        mesh=_mesh(),
        scratch_types=[pltpu.VMEM((CH,), jnp.int32),
                       pltpu.VMEM((CH, PW), jnp.float32),
                       pltpu.VMEM((RZ, PW), jnp.float32),
                       pltpu.SemaphoreType.DMA,
                       pltpu.VMEM_SHARED((N, PW), jnp.float32)],
        compiler_params=pltpu.CompilerParams(use_tc_tiling_on_sc=False),
    )

_NF = 1000


def _final_body(acc_ref, out_ref):
    a = acc_ref[...]
    s = a[0] + a[1]
    cnt = s[:, OUT:OUT + 1]
    out_ref[...] = s[:, :OUT] / jnp.maximum(cnt, 1.0)


def _finalize(acc):
    return pl.pallas_call(
        _final_body,
        grid=(N // _NF,),
        in_specs=[pl.BlockSpec((NC, _NF, PW), lambda i: (0, i, 0))],
        out_specs=pl.BlockSpec((_NF, OUT), lambda i: (i, 0)),
        out_shape=jax.ShapeDtypeStruct((N, OUT), jnp.float32),
    )(acc)


def kernel(x, edge_index, edge_attr, W1, b1, W2, b2, W3, b3, W4, b4):
    ei = edge_index.astype(jnp.int32)
    src = ei[0]
    dst = ei[1]
    wat = W1[:, :D].T
    wbt = W1[:, D:2 * D].T
    w1ct = W1[:, 2 * D:].T
    xa, xb = _proj(x, wat, wbt)
    ga, gb = _gather()(xa, xb, src, dst)
    msg = _edge_mlp(ga, gb, edge_attr, w1ct, b1.reshape(1, H),
                    W2.T, b2.reshape(1, H), W3.T, b3.reshape(1, H),
                    W4.T, b4.reshape(1, OUT))
    zrows = jnp.zeros((RZ, PW), jnp.float32)
    acc = _scatter()(msg, dst, zrows)
    return _finalize(acc)


# speedup vs baseline: 4.0670x; 1.8882x over previous
import functools

import jax
import jax.numpy as jnp
from jax import lax
from jax.experimental import pallas as pl
from jax.experimental.pallas import tpu as pltpu
from jax.experimental.pallas import tpu_sc as plsc

N = 10000
E = 320000
D = 128
DE = 16
H = 64
OUT = 128

NC, NS = 2, 16
NW = NC * NS
EPW = E // NW
CH = 80
NCH = EPW // CH
RPT = 624
RZ = 104
CR = 80


@functools.cache
def _mesh():
    return plsc.VectorSubcoreMesh(core_axis_name="c", subcore_axis_name="s",
                                  num_cores=NC, num_subcores=NS)


_NB = 1000


def _proj_body(x_ref, w_ref, o_ref):
    o_ref[...] = jnp.dot(x_ref[...], w_ref[...],
                         preferred_element_type=jnp.float32)


def _proj(x, w1abt):
    return pl.pallas_call(
        _proj_body,
        grid=(N // _NB,),
        in_specs=[pl.BlockSpec((_NB, D), lambda i: (i, 0)),
                  pl.BlockSpec((D, D), lambda i: (0, 0))],
        out_specs=pl.BlockSpec((_NB, D), lambda i: (i, 0)),
        out_shape=jax.ShapeDtypeStruct((N, D), jnp.float32),
    )(x, w1abt)


def _gather_body(tab, inter, s_out,
                 ia0, ib0, gbuf0, ia1, ib1, gbuf1,
                 sa0, sb0, ga0, gb0, ws0, sa1, sb1, ga1, gb1, ws1):
    wid = lax.axis_index("s") * NC + lax.axis_index("c")
    base = wid * EPW
    P = ((ia0, ib0, gbuf0, sa0, sb0, ga0, gb0, ws0),
         (ia1, ib1, gbuf1, sa1, sb1, ga1, gb1, ws1))

    def idx_issue(c, p):
        ia, ib, _, sa, sb, *_ = P[p]
        off = 2 * (base + c * CH)
        pltpu.async_copy(inter.at[pl.ds(off, CH)], ia, sa)
        pltpu.async_copy(inter.at[pl.ds(off + CH, CH)], ib, sb)

    def idx_wait(p):
        ia, ib, _, sa, sb, *_ = P[p]
        pltpu.make_async_copy(inter.at[pl.ds(0, CH)], ia, sa).wait()
        pltpu.make_async_copy(inter.at[pl.ds(0, CH)], ib, sb).wait()

    def gather_issue(p):
        ia, ib, gb_, _, _, ga, gb2, _ = P[p]
        pltpu.async_copy(tab.at[ia], gb_.at[pl.ds(0, CH)], ga)
        pltpu.async_copy(tab.at[ib], gb_.at[pl.ds(CH, CH)], gb2)

    def gather_wait(p):
        ia, ib, gb_, _, _, ga, gb2, _ = P[p]
        pltpu.make_async_copy(tab.at[ia], gb_.at[pl.ds(0, CH)], ga).wait()
        pltpu.make_async_copy(tab.at[ib], gb_.at[pl.ds(CH, CH)], gb2).wait()

    def write_issue(c, p):
        gb_ = P[p][2]
        off = 2 * (base + c * CH)
        pltpu.async_copy(gb_, s_out.at[pl.ds(off, 2 * CH)], P[p][7])

    def write_wait(p):
        gb_ = P[p][2]
        pltpu.make_async_copy(gb_, s_out.at[pl.ds(0, 2 * CH)], P[p][7]).wait()

    idx_issue(0, 0)
    idx_issue(1, 1)
    idx_wait(0)
    gather_issue(0)
    idx_wait(1)
    gather_issue(1)
    gather_wait(0)
    write_issue(0, 0)
    idx_issue(2, 0)

    def body(k, carry):
        c = 2 * k
        write_wait(0)
        idx_wait(0)
        gather_issue(0)
        gather_wait(1)
        write_issue(c - 1, 1)
        idx_issue(c + 1, 1)
        write_wait(1)
        idx_wait(1)
        gather_issue(1)
        gather_wait(0)
        write_issue(c, 0)
        idx_issue(c + 2, 0)
        return carry

    lax.fori_loop(1, (NCH - 1) // 2, body, 0)
    write_wait(0)
    idx_wait(0)
    gather_issue(0)
    gather_wait(1)
    write_issue(NCH - 2, 1)
    gather_wait(0)
    write_issue(NCH - 1, 0)
    write_wait(1)
    write_wait(0)


@functools.cache
def _gather():
    return pl.kernel(
        _gather_body,
        out_type=jax.ShapeDtypeStruct((2 * E, H), jnp.float32),
        mesh=_mesh(),
        scratch_types=[pltpu.VMEM((CH,), jnp.int32),
                       pltpu.VMEM((CH,), jnp.int32),
                       pltpu.VMEM((2 * CH, H), jnp.float32),
                       pltpu.VMEM((CH,), jnp.int32),
                       pltpu.VMEM((CH,), jnp.int32),
                       pltpu.VMEM((2 * CH, H), jnp.float32)]
                      + [pltpu.SemaphoreType.DMA] * 10,
        compiler_params=pltpu.CompilerParams(needs_layout_passes=False,
                                             use_tc_tiling_on_sc=False),
    )


_BE = 2000


def _mlp_body(s, ea, w1ct, b1, w2t, b2, w3t, b3, w4t, b4, out):
    h = (s[:, :H] + s[:, H:]
         + jnp.dot(ea[...], w1ct[...], preferred_element_type=jnp.float32)
         + b1[...])
    h = jnp.maximum(h, 0.0)
    h = jnp.maximum(
        jnp.dot(h, w2t[...], preferred_element_type=jnp.float32) + b2[...], 0.0)
    h = jnp.maximum(
        jnp.dot(h, w3t[...], preferred_element_type=jnp.float32) + b3[...], 0.0)
    out[...] = jnp.dot(h, w4t[...], preferred_element_type=jnp.float32) + b4[...]


def _edge_mlp(s, ea, w1ct, b1, w2t, b2, w3t, b3, w4t, b4):
    zero = lambda i: (0, 0)
    return pl.pallas_call(
        _mlp_body,
        grid=(E // _BE,),
        in_specs=[pl.BlockSpec((_BE, D), lambda i: (i, 0)),
                  pl.BlockSpec((_BE, DE), lambda i: (i, 0)),
                  pl.BlockSpec((DE, H), zero),
                  pl.BlockSpec((1, H), zero),
                  pl.BlockSpec((H, H), zero),
                  pl.BlockSpec((1, H), zero),
                  pl.BlockSpec((H, H), zero),
                  pl.BlockSpec((1, H), zero),
                  pl.BlockSpec((H, OUT), zero),
                  pl.BlockSpec((1, OUT), zero)],
        out_specs=pl.BlockSpec((_BE, OUT), lambda i: (i, 0)),
        out_shape=jax.ShapeDtypeStruct((E, OUT), jnp.float32),
    )(s, ea, w1ct, b1, w2t, b2, w3t, b3, w4t, b4)


def _scatter_body(msg, dst, zrows, sums, cnt,
                  didx, mbuf, didx1, mbuf1, zbuf, iota_v, cntloc,
                  li0, lm0, sc0, li1, lm1, sc1, acc, cntsh):
    cid = lax.axis_index("c")
    sid = lax.axis_index("s")
    pltpu.sync_copy(zrows, zbuf)
    for k in range(RPT // RZ):
        pltpu.sync_copy(zbuf, acc.at[pl.ds(sid * RPT + k * RZ, RZ)])
    zv = jnp.zeros((16,), jnp.float32)

    def zrow(rr, c):
        for j in range(128 // 16):
            cntloc[rr, pl.ds(j * 16, 16)] = zv
        return c

    lax.fori_loop(0, CR, zrow, 0)

    @pl.when(sid == NS - 1)
    def _():
        pltpu.sync_copy(zbuf.at[pl.ds(0, N - NS * RPT)],
                        acc.at[pl.ds(NS * RPT, N - NS * RPT)])

    @pl.when(sid == 0)
    def _():
        pltpu.sync_copy(zbuf.at[pl.ds(0, CR)], cntsh)

    for j in range(CR // 16):
        iota_v[pl.ds(j * 16, 16)] = lax.iota(jnp.int32, 16) + j * 16
    plsc.subcore_barrier()

    wid = sid * NC + cid
    base = wid * EPW
    ones16 = jnp.ones((16,), jnp.float32)
    P = ((didx, mbuf, li0, lm0, sc0), (didx1, mbuf1, li1, lm1, sc1))

    def load_issue(c, p):
        di, mb, li, lm, _ = P[p]
        off = base + c * CH
        pltpu.async_copy(dst.at[pl.ds(off, CH)], di, li)
        pltpu.async_copy(msg.at[pl.ds(off, CH)], mb, lm)

    def load_wait(p):
        di, mb, li, lm, _ = P[p]
        pltpu.make_async_copy(dst.at[pl.ds(0, CH)], di, li).wait()
        pltpu.make_async_copy(msg.at[pl.ds(0, CH)], mb, lm).wait()

    def scatter_issue(p):
        di, mb, _, _, sc = P[p]
        pltpu.async_copy(mb, acc.at[di], sc, add=True)

    def scatter_wait(p):
        di, mb, _, _, sc = P[p]
        pltpu.make_async_copy(mb, acc.at[di], sc).wait()

    def hist(p):
        di = P[p][0]
        for j in range(CH // 16):
            dv = di[pl.ds(j * 16, 16)]
            r = lax.shift_right_logical(dv, 7)
            c = lax.bitwise_and(dv, 127)
            plsc.addupdate_scatter(cntloc, [r, c], ones16)

    load_issue(0, 0)
    load_wait(0)
    scatter_issue(0)
    hist(0)
    load_issue(1, 1)
    load_wait(1)
    scatter_issue(1)
    hist(1)
    scatter_wait(0)
    load_issue(2, 0)

    def step(k, carry):
        c = 2 * k
        load_wait(0)
        scatter_issue(0)
        hist(0)
        scatter_wait(1)
        load_issue(c + 1, 1)
        load_wait(1)
        scatter_issue(1)
        hist(1)
        scatter_wait(0)
        load_issue(c + 2, 0)
        return carry

    lax.fori_loop(1, (NCH - 1) // 2, step, 0)
    load_wait(0)
    scatter_issue(0)
    hist(0)
    scatter_wait(1)
    scatter_wait(0)
    pltpu.sync_copy(cntloc, cntsh.at[iota_v], add=True)
    plsc.subcore_barrier()
    for k in range(RPT // RZ):
        r0 = sid * RPT + k * RZ
        pltpu.sync_copy(acc.at[pl.ds(r0, RZ)], zbuf)
        pltpu.sync_copy(zbuf, sums.at[cid, pl.ds(r0, RZ)])

    @pl.when(sid == NS - 1)
    def _():
        tail = N - NS * RPT
        pltpu.sync_copy(acc.at[pl.ds(NS * RPT, tail)], zbuf.at[pl.ds(0, tail)])
        pltpu.sync_copy(zbuf.at[pl.ds(0, tail)],
                        sums.at[cid, pl.ds(NS * RPT, tail)])

    @pl.when(sid == 0)
    def _():
        pltpu.sync_copy(cntsh, cntloc)
        pltpu.sync_copy(cntloc, cnt.at[cid])


@functools.cache
def _scatter():
    return pl.kernel(
        _scatter_body,
        out_type=(jax.ShapeDtypeStruct((NC, N, OUT), jnp.float32),
                  jax.ShapeDtypeStruct((NC, CR, 128), jnp.float32)),
        mesh=_mesh(),
        scratch_types=[pltpu.VMEM((CH,), jnp.int32),
                       pltpu.VMEM((CH, OUT), jnp.float32),
                       pltpu.VMEM((CH,), jnp.int32),
                       pltpu.VMEM((CH, OUT), jnp.float32),
                       pltpu.VMEM((RZ, OUT), jnp.float32),
                       pltpu.VMEM((CR,), jnp.int32),
                       pltpu.VMEM((CR, 128), jnp.float32)]
                      + [pltpu.SemaphoreType.DMA] * 6
                      + [pltpu.VMEM_SHARED((N, OUT), jnp.float32),
                         pltpu.VMEM_SHARED((CR, 128), jnp.float32)],
        compiler_params=pltpu.CompilerParams(needs_layout_passes=False),
    )


_NF = 1000


def _final_body(sums_ref, cnt_ref, out_ref):
    a = sums_ref[...]
    s = a[0] + a[1]
    c = cnt_ref[...]
    c2 = c[0] + c[1]
    out_ref[...] = s / jnp.maximum(c2, 1.0)


def _finalize(sums, cnt_col):
    return pl.pallas_call(
        _final_body,
        grid=(N // _NF,),
        in_specs=[pl.BlockSpec((NC, _NF, OUT), lambda i: (0, i, 0)),
                  pl.BlockSpec((NC, _NF, 1), lambda i: (0, i, 0))],
        out_specs=pl.BlockSpec((_NF, OUT), lambda i: (i, 0)),
        out_shape=jax.ShapeDtypeStruct((N, OUT), jnp.float32),
    )(sums, cnt_col)


def kernel(x, edge_index, edge_attr, W1, b1, W2, b2, W3, b3, W4, b4):
    ei = edge_index.astype(jnp.int32)
    src = ei[0]
    dst = ei[1]
    w1abt = W1[:, :2 * D].T
    w1abt = jnp.concatenate([w1abt[:D], w1abt[D:]], axis=1)
    w1ct = W1[:, 2 * D:].T
    tab = _proj(x, w1abt)
    inter = jnp.stack([src * 2, dst * 2 + 1], axis=1).reshape(2 * E)
    s_arr = _gather()(tab.reshape(2 * N, H), inter).reshape(E, D)
    msg = _edge_mlp(s_arr, edge_attr, w1ct, b1.reshape(1, H),
                    W2.T, b2.reshape(1, H), W3.T, b3.reshape(1, H),
                    W4.T, b4.reshape(1, OUT))
    zrows = jnp.zeros((RZ, OUT), jnp.float32)
    sums, cnt = _scatter()(msg, dst, zrows)
    cnt_col = cnt.reshape(NC, CR * 128)[:, :N].reshape(NC, N, 1)
    return _finalize(sums, cnt_col)


# speedup vs baseline: 4.2123x; 1.0357x over previous
import functools

import jax
import jax.numpy as jnp
from jax import lax
from jax.experimental import pallas as pl
from jax.experimental.pallas import tpu as pltpu
from jax.experimental.pallas import tpu_sc as plsc

N = 10000
E = 320000
D = 128
DE = 16
H = 64
OUT = 128

NC, NS = 2, 16
NW = NC * NS
EPW = E // NW
CH = 80
NCH = EPW // CH
RPT = 624
RZ = 104
CR = 80


@functools.cache
def _mesh():
    return plsc.VectorSubcoreMesh(core_axis_name="c", subcore_axis_name="s",
                                  num_cores=NC, num_subcores=NS)


_NB = 2000


def _proj_body(x_ref, w_ref, o_ref):
    o_ref[...] = jnp.dot(x_ref[...], w_ref[...],
                         preferred_element_type=jnp.float32)


def _proj(x, w1abt):
    return pl.pallas_call(
        _proj_body,
        grid=(N // _NB,),
        in_specs=[pl.BlockSpec((_NB, D), lambda i: (i, 0)),
                  pl.BlockSpec((D, D), lambda i: (0, 0))],
        out_specs=pl.BlockSpec((_NB, D), lambda i: (i, 0)),
        out_shape=jax.ShapeDtypeStruct((N, D), jnp.float32),
    )(x, w1abt)


def _gather_body(tab, inter, s_out,
                 ia0, ib0, gbuf0, ia1, ib1, gbuf1,
                 sa0, sb0, ga0, gb0, ws0, sa1, sb1, ga1, gb1, ws1):
    wid = lax.axis_index("s") * NC + lax.axis_index("c")
    base = wid * EPW
    P = ((ia0, ib0, gbuf0, sa0, sb0, ga0, gb0, ws0),
         (ia1, ib1, gbuf1, sa1, sb1, ga1, gb1, ws1))

    def idx_issue(c, p):
        ia, ib, _, sa, sb, *_ = P[p]
        off = 2 * (base + c * CH)
        pltpu.async_copy(inter.at[pl.ds(off, CH)], ia, sa)
        pltpu.async_copy(inter.at[pl.ds(off + CH, CH)], ib, sb)

    def idx_wait(p):
        ia, ib, _, sa, sb, *_ = P[p]
        pltpu.make_async_copy(inter.at[pl.ds(0, CH)], ia, sa).wait()
        pltpu.make_async_copy(inter.at[pl.ds(0, CH)], ib, sb).wait()

    def gather_issue(p):
        ia, ib, gb_, _, _, ga, gb2, _ = P[p]
        pltpu.async_copy(tab.at[ia], gb_.at[pl.ds(0, CH)], ga)
        pltpu.async_copy(tab.at[ib], gb_.at[pl.ds(CH, CH)], gb2)

    def gather_wait(p):
        ia, ib, gb_, _, _, ga, gb2, _ = P[p]
        pltpu.make_async_copy(tab.at[ia], gb_.at[pl.ds(0, CH)], ga).wait()
        pltpu.make_async_copy(tab.at[ib], gb_.at[pl.ds(CH, CH)], gb2).wait()

    def write_issue(c, p):
        gb_ = P[p][2]
        off = 2 * (base + c * CH)
        pltpu.async_copy(gb_, s_out.at[pl.ds(off, 2 * CH)], P[p][7])

    def write_wait(p):
        gb_ = P[p][2]
        pltpu.make_async_copy(gb_, s_out.at[pl.ds(0, 2 * CH)], P[p][7]).wait()

    idx_issue(0, 0)
    idx_issue(1, 1)
    idx_wait(0)
    gather_issue(0)
    idx_wait(1)
    gather_issue(1)
    gather_wait(0)
    write_issue(0, 0)
    idx_issue(2, 0)

    def body(k, carry):
        c = 2 * k
        write_wait(0)
        idx_wait(0)
        gather_issue(0)
        gather_wait(1)
        write_issue(c - 1, 1)
        idx_issue(c + 1, 1)
        write_wait(1)
        idx_wait(1)
        gather_issue(1)
        gather_wait(0)
        write_issue(c, 0)
        idx_issue(c + 2, 0)
        return carry

    lax.fori_loop(1, (NCH - 1) // 2, body, 0)
    write_wait(0)
    idx_wait(0)
    gather_issue(0)
    gather_wait(1)
    write_issue(NCH - 2, 1)
    gather_wait(0)
    write_issue(NCH - 1, 0)
    write_wait(1)
    write_wait(0)


@functools.cache
def _gather():
    return pl.kernel(
        _gather_body,
        out_type=jax.ShapeDtypeStruct((2 * E, H), jnp.float32),
        mesh=_mesh(),
        scratch_types=[pltpu.VMEM((CH,), jnp.int32),
                       pltpu.VMEM((CH,), jnp.int32),
                       pltpu.VMEM((2 * CH, H), jnp.float32),
                       pltpu.VMEM((CH,), jnp.int32),
                       pltpu.VMEM((CH,), jnp.int32),
                       pltpu.VMEM((2 * CH, H), jnp.float32)]
                      + [pltpu.SemaphoreType.DMA] * 10,
        compiler_params=pltpu.CompilerParams(needs_layout_passes=False,
                                             use_tc_tiling_on_sc=False),
    )


_BE = 5000


def _mlp_body(s, ea, w1ct, b1, w2t, b2, w3t, b3, w4t, b4, out):
    h = (s[:, :H] + s[:, H:]
         + jnp.dot(ea[...], w1ct[...], preferred_element_type=jnp.float32)
         + b1[...])
    h = jnp.maximum(h, 0.0)
    h = jnp.maximum(
        jnp.dot(h, w2t[...], preferred_element_type=jnp.float32) + b2[...], 0.0)
    h = jnp.maximum(
        jnp.dot(h, w3t[...], preferred_element_type=jnp.float32) + b3[...], 0.0)
    out[...] = jnp.dot(h, w4t[...], preferred_element_type=jnp.float32) + b4[...]


def _edge_mlp(s, ea, w1ct, b1, w2t, b2, w3t, b3, w4t, b4):
    zero = lambda i: (0, 0)
    return pl.pallas_call(
        _mlp_body,
        grid=(E // _BE,),
        in_specs=[pl.BlockSpec((_BE, D), lambda i: (i, 0)),
                  pl.BlockSpec((_BE, DE), lambda i: (i, 0)),
                  pl.BlockSpec((DE, H), zero),
                  pl.BlockSpec((1, H), zero),
                  pl.BlockSpec((H, H), zero),
                  pl.BlockSpec((1, H), zero),
                  pl.BlockSpec((H, H), zero),
                  pl.BlockSpec((1, H), zero),
                  pl.BlockSpec((H, OUT), zero),
                  pl.BlockSpec((1, OUT), zero)],
        out_specs=pl.BlockSpec((_BE, OUT), lambda i: (i, 0)),
        out_shape=jax.ShapeDtypeStruct((E, OUT), jnp.float32),
    )(s, ea, w1ct, b1, w2t, b2, w3t, b3, w4t, b4)


def _scatter_body(msg, dst, zrows, sums, cnt,
                  didx, mbuf, didx1, mbuf1, zbuf, iota_v, cntloc,
                  li0, lm0, sc0, li1, lm1, sc1, acc, cntsh):
    cid = lax.axis_index("c")
    sid = lax.axis_index("s")
    pltpu.sync_copy(zrows, zbuf)
    for k in range(RPT // RZ):
        pltpu.sync_copy(zbuf, acc.at[pl.ds(sid * RPT + k * RZ, RZ)])
    zv = jnp.zeros((16,), jnp.float32)

    def zrow(rr, c):
        for j in range(128 // 16):
            cntloc[rr, pl.ds(j * 16, 16)] = zv
        return c

    lax.fori_loop(0, CR, zrow, 0)

    @pl.when(sid == NS - 1)
    def _():
        pltpu.sync_copy(zbuf.at[pl.ds(0, N - NS * RPT)],
                        acc.at[pl.ds(NS * RPT, N - NS * RPT)])

    @pl.when(sid == 0)
    def _():
        pltpu.sync_copy(zbuf.at[pl.ds(0, CR)], cntsh)

    for j in range(CR // 16):
        iota_v[pl.ds(j * 16, 16)] = lax.iota(jnp.int32, 16) + j * 16
    plsc.subcore_barrier()

    wid = sid * NC + cid
    base = wid * EPW
    ones16 = jnp.ones((16,), jnp.float32)
    P = ((didx, mbuf, li0, lm0, sc0), (didx1, mbuf1, li1, lm1, sc1))

    def load_issue(c, p):
        di, mb, li, lm, _ = P[p]
        off = base + c * CH
        pltpu.async_copy(dst.at[pl.ds(off, CH)], di, li)
        pltpu.async_copy(msg.at[pl.ds(off, CH)], mb, lm)

    def load_wait(p):
        di, mb, li, lm, _ = P[p]
        pltpu.make_async_copy(dst.at[pl.ds(0, CH)], di, li).wait()
        pltpu.make_async_copy(msg.at[pl.ds(0, CH)], mb, lm).wait()

    def scatter_issue(p):
        di, mb, _, _, sc = P[p]
        pltpu.async_copy(mb, acc.at[di], sc, add=True)

    def scatter_wait(p):
        di, mb, _, _, sc = P[p]
        pltpu.make_async_copy(mb, acc.at[di], sc).wait()

    def hist(p):
        di = P[p][0]
        for j in range(CH // 16):
            dv = di[pl.ds(j * 16, 16)]
            r = lax.shift_right_logical(dv, 7)
            c = lax.bitwise_and(dv, 127)
            plsc.addupdate_scatter(cntloc, [r, c], ones16)

    load_issue(0, 0)
    load_wait(0)
    scatter_issue(0)
    hist(0)
    load_issue(1, 1)
    load_wait(1)
    scatter_issue(1)
    hist(1)
    scatter_wait(0)
    load_issue(2, 0)

    def step(k, carry):
        c = 2 * k
        load_wait(0)
        scatter_issue(0)
        hist(0)
        scatter_wait(1)
        load_issue(c + 1, 1)
        load_wait(1)
        scatter_issue(1)
        hist(1)
        scatter_wait(0)
        load_issue(c + 2, 0)
        return carry

    lax.fori_loop(1, (NCH - 1) // 2, step, 0)
    load_wait(0)
    scatter_issue(0)
    hist(0)
    scatter_wait(1)
    scatter_wait(0)
    pltpu.sync_copy(cntloc, cntsh.at[iota_v], add=True)
    plsc.subcore_barrier()
    for k in range(RPT // RZ):
        r0 = sid * RPT + k * RZ
        pltpu.sync_copy(acc.at[pl.ds(r0, RZ)], zbuf)
        pltpu.sync_copy(zbuf, sums.at[cid, pl.ds(r0, RZ)])

    @pl.when(sid == NS - 1)
    def _():
        tail = N - NS * RPT
        pltpu.sync_copy(acc.at[pl.ds(NS * RPT, tail)], zbuf.at[pl.ds(0, tail)])
        pltpu.sync_copy(zbuf.at[pl.ds(0, tail)],
                        sums.at[cid, pl.ds(NS * RPT, tail)])

    @pl.when(sid == 0)
    def _():
        pltpu.sync_copy(cntsh, cntloc)
        pltpu.sync_copy(cntloc, cnt.at[cid])


@functools.cache
def _scatter():
    return pl.kernel(
        _scatter_body,
        out_type=(jax.ShapeDtypeStruct((NC, N, OUT), jnp.float32),
                  jax.ShapeDtypeStruct((NC, CR, 128), jnp.float32)),
        mesh=_mesh(),
        scratch_types=[pltpu.VMEM((CH,), jnp.int32),
                       pltpu.VMEM((CH, OUT), jnp.float32),
                       pltpu.VMEM((CH,), jnp.int32),
                       pltpu.VMEM((CH, OUT), jnp.float32),
                       pltpu.VMEM((RZ, OUT), jnp.float32),
                       pltpu.VMEM((CR,), jnp.int32),
                       pltpu.VMEM((CR, 128), jnp.float32)]
                      + [pltpu.SemaphoreType.DMA] * 6
                      + [pltpu.VMEM_SHARED((N, OUT), jnp.float32),
                         pltpu.VMEM_SHARED((CR, 128), jnp.float32)],
        compiler_params=pltpu.CompilerParams(needs_layout_passes=False),
    )


_NF = 1000


def _final_body(sums_ref, cnt_ref, out_ref):
    a = sums_ref[...]
    s = a[0] + a[1]
    c = cnt_ref[...]
    c2 = c[0] + c[1]
    out_ref[...] = s / jnp.maximum(c2, 1.0)


def _finalize(sums, cnt_col):
    return pl.pallas_call(
        _final_body,
        grid=(N // _NF,),
        in_specs=[pl.BlockSpec((NC, _NF, OUT), lambda i: (0, i, 0)),
                  pl.BlockSpec((NC, _NF, 1), lambda i: (0, i, 0))],
        out_specs=pl.BlockSpec((_NF, OUT), lambda i: (i, 0)),
        out_shape=jax.ShapeDtypeStruct((N, OUT), jnp.float32),
    )(sums, cnt_col)


def kernel(x, edge_index, edge_attr, W1, b1, W2, b2, W3, b3, W4, b4):
    ei = edge_index.astype(jnp.int32)
    src = ei[0]
    dst = ei[1]
    w1abt = W1[:, :2 * D].T
    w1abt = jnp.concatenate([w1abt[:D], w1abt[D:]], axis=1)
    w1ct = W1[:, 2 * D:].T
    tab = _proj(x, w1abt)
    inter = jnp.stack([src * 2, dst * 2 + 1], axis=1).reshape(2 * E)
    s_arr = _gather()(tab.reshape(2 * N, H), inter).reshape(E, D)
    msg = _edge_mlp(s_arr, edge_attr, w1ct, b1.reshape(1, H),
                    W2.T, b2.reshape(1, H), W3.T, b3.reshape(1, H),
                    W4.T, b4.reshape(1, OUT))
    zrows = jnp.zeros((RZ, OUT), jnp.float32)
    sums, cnt = _scatter()(msg, dst, zrows)
    cnt_col = cnt.reshape(NC, CR * 128)[:, :N].reshape(NC, N, 1)
    return _finalize(sums, cnt_col)


# speedup vs baseline: 4.4551x; 1.0576x over previous
import functools

import jax
import jax.numpy as jnp
from jax import lax
from jax.experimental import pallas as pl
from jax.experimental.pallas import tpu as pltpu
from jax.experimental.pallas import tpu_sc as plsc

N = 10000
E = 320000
D = 128
DE = 16
H = 64
OUT = 128

NC, NS = 2, 16
NW = NC * NS
EPW = E // NW
CH = 80
NCH = EPW // CH
RPT = 624
RZ = 104
CR = 80


@functools.cache
def _mesh():
    return plsc.VectorSubcoreMesh(core_axis_name="c", subcore_axis_name="s",
                                  num_cores=NC, num_subcores=NS)


_NB = 2000


def _proj_body(x_ref, w_ref, o_ref):
    o_ref[...] = jnp.dot(x_ref[...], w_ref[...],
                         preferred_element_type=jnp.float32)


def _proj(x, w1abt):
    return pl.pallas_call(
        _proj_body,
        grid=(N // _NB,),
        in_specs=[pl.BlockSpec((_NB, D), lambda i: (i, 0)),
                  pl.BlockSpec((D, D), lambda i: (0, 0))],
        out_specs=pl.BlockSpec((_NB, D), lambda i: (i, 0)),
        out_shape=jax.ShapeDtypeStruct((N, D), jnp.float32),
    )(x, w1abt)


def _gather_body(tab, inter, s_out,
                 ia0, ib0, gbuf0, ia1, ib1, gbuf1,
                 sa0, sb0, ga0, gb0, ws0, sa1, sb1, ga1, gb1, ws1):
    wid = lax.axis_index("s") * NC + lax.axis_index("c")
    base = wid * EPW
    P = ((ia0, ib0, gbuf0, sa0, sb0, ga0, gb0, ws0),
         (ia1, ib1, gbuf1, sa1, sb1, ga1, gb1, ws1))

    def idx_issue(c, p):
        ia, ib, _, sa, sb, *_ = P[p]
        off = 2 * (base + c * CH)
        pltpu.async_copy(inter.at[pl.ds(off, CH)], ia, sa)
        pltpu.async_copy(inter.at[pl.ds(off + CH, CH)], ib, sb)

    def idx_wait(p):
        ia, ib, _, sa, sb, *_ = P[p]
        pltpu.make_async_copy(inter.at[pl.ds(0, CH)], ia, sa).wait()
        pltpu.make_async_copy(inter.at[pl.ds(0, CH)], ib, sb).wait()

    def gather_issue(p):
        ia, ib, gb_, _, _, ga, gb2, _ = P[p]
        pltpu.async_copy(tab.at[ia], gb_.at[pl.ds(0, CH)], ga)
        pltpu.async_copy(tab.at[ib], gb_.at[pl.ds(CH, CH)], gb2)

    def gather_wait(p):
        ia, ib, gb_, _, _, ga, gb2, _ = P[p]
        pltpu.make_async_copy(tab.at[ia], gb_.at[pl.ds(0, CH)], ga).wait()
        pltpu.make_async_copy(tab.at[ib], gb_.at[pl.ds(CH, CH)], gb2).wait()

    def write_issue(c, p):
        gb_ = P[p][2]
        off = 2 * (base + c * CH)
        pltpu.async_copy(gb_, s_out.at[pl.ds(off, 2 * CH)], P[p][7])

    def write_wait(p):
        gb_ = P[p][2]
        pltpu.make_async_copy(gb_, s_out.at[pl.ds(0, 2 * CH)], P[p][7]).wait()

    idx_issue(0, 0)
    idx_issue(1, 1)
    idx_wait(0)
    gather_issue(0)
    idx_wait(1)
    gather_issue(1)
    gather_wait(0)
    write_issue(0, 0)
    idx_issue(2, 0)

    def body(k, carry):
        c = 2 * k
        write_wait(0)
        idx_wait(0)
        gather_issue(0)
        gather_wait(1)
        write_issue(c - 1, 1)
        idx_issue(c + 1, 1)
        write_wait(1)
        idx_wait(1)
        gather_issue(1)
        gather_wait(0)
        write_issue(c, 0)
        idx_issue(c + 2, 0)
        return carry

    lax.fori_loop(1, (NCH - 1) // 2, body, 0)
    write_wait(0)
    idx_wait(0)
    gather_issue(0)
    gather_wait(1)
    write_issue(NCH - 2, 1)
    gather_wait(0)
    write_issue(NCH - 1, 0)
    write_wait(1)
    write_wait(0)


@functools.cache
def _gather():
    return pl.kernel(
        _gather_body,
        out_type=jax.ShapeDtypeStruct((2 * E, H), jnp.float32),
        mesh=_mesh(),
        scratch_types=[pltpu.VMEM((CH,), jnp.int32),
                       pltpu.VMEM((CH,), jnp.int32),
                       pltpu.VMEM((2 * CH, H), jnp.float32),
                       pltpu.VMEM((CH,), jnp.int32),
                       pltpu.VMEM((CH,), jnp.int32),
                       pltpu.VMEM((2 * CH, H), jnp.float32)]
                      + [pltpu.SemaphoreType.DMA] * 10,
        compiler_params=pltpu.CompilerParams(needs_layout_passes=False,
                                             use_tc_tiling_on_sc=False),
    )


_BE = 10000


def _mlp_body(s, ea, w1ct, b1, w2t, b2, w3t, b3, w4t, b4, out):
    h = (s[:, :H] + s[:, H:]
         + jnp.dot(ea[...], w1ct[...], preferred_element_type=jnp.float32)
         + b1[...])
    h = jnp.maximum(h, 0.0)
    h = jnp.maximum(
        jnp.dot(h, w2t[...], preferred_element_type=jnp.float32) + b2[...], 0.0)
    h = jnp.maximum(
        jnp.dot(h, w3t[...], preferred_element_type=jnp.float32) + b3[...], 0.0)
    out[...] = jnp.dot(h, w4t[...], preferred_element_type=jnp.float32) + b4[...]


def _edge_mlp(s, ea, w1ct, b1, w2t, b2, w3t, b3, w4t, b4):
    zero = lambda i: (0, 0)
    return pl.pallas_call(
        _mlp_body,
        grid=(E // _BE,),
        in_specs=[pl.BlockSpec((_BE, D), lambda i: (i, 0)),
                  pl.BlockSpec((_BE, DE), lambda i: (i, 0)),
                  pl.BlockSpec((DE, H), zero),
                  pl.BlockSpec((1, H), zero),
                  pl.BlockSpec((H, H), zero),
                  pl.BlockSpec((1, H), zero),
                  pl.BlockSpec((H, H), zero),
                  pl.BlockSpec((1, H), zero),
                  pl.BlockSpec((H, OUT), zero),
                  pl.BlockSpec((1, OUT), zero)],
        out_specs=pl.BlockSpec((_BE, OUT), lambda i: (i, 0)),
        out_shape=jax.ShapeDtypeStruct((E, OUT), jnp.float32),
    )(s, ea, w1ct, b1, w2t, b2, w3t, b3, w4t, b4)


def _scatter_body(msg, dst, zrows, sums, cnt,
                  didx, mbuf, didx1, mbuf1, zbuf, iota_v, cntloc,
                  li0, lm0, sc0, li1, lm1, sc1, acc, cntsh):
    cid = lax.axis_index("c")
    sid = lax.axis_index("s")
    pltpu.sync_copy(zrows, zbuf)
    for k in range(RPT // RZ):
        pltpu.sync_copy(zbuf, acc.at[pl.ds(sid * RPT + k * RZ, RZ)])
    zv = jnp.zeros((16,), jnp.float32)

    def zrow(rr, c):
        for j in range(128 // 16):
            cntloc[rr, pl.ds(j * 16, 16)] = zv
        return c

    lax.fori_loop(0, CR, zrow, 0)

    @pl.when(sid == NS - 1)
    def _():
        pltpu.sync_copy(zbuf.at[pl.ds(0, N - NS * RPT)],
                        acc.at[pl.ds(NS * RPT, N - NS * RPT)])

    @pl.when(sid == 0)
    def _():
        pltpu.sync_copy(zbuf.at[pl.ds(0, CR)], cntsh)

    for j in range(CR // 16):
        iota_v[pl.ds(j * 16, 16)] = lax.iota(jnp.int32, 16) + j * 16
    plsc.subcore_barrier()

    wid = sid * NC + cid
    base = wid * EPW
    ones16 = jnp.ones((16,), jnp.float32)
    P = ((didx, mbuf, li0, lm0, sc0), (didx1, mbuf1, li1, lm1, sc1))

    def load_issue(c, p):
        di, mb, li, lm, _ = P[p]
        off = base + c * CH
        pltpu.async_copy(dst.at[pl.ds(off, CH)], di, li)
        pltpu.async_copy(msg.at[pl.ds(off, CH)], mb, lm)

    def load_wait(p):
        di, mb, li, lm, _ = P[p]
        pltpu.make_async_copy(dst.at[pl.ds(0, CH)], di, li).wait()
        pltpu.make_async_copy(msg.at[pl.ds(0, CH)], mb, lm).wait()

    def scatter_issue(p):
        di, mb, _, _, sc = P[p]
        pltpu.async_copy(mb, acc.at[di], sc, add=True)

    def scatter_wait(p):
        di, mb, _, _, sc = P[p]
        pltpu.make_async_copy(mb, acc.at[di], sc).wait()

    def hist(p):
        di = P[p][0]
        for j in range(CH // 16):
            dv = di[pl.ds(j * 16, 16)]
            r = lax.shift_right_logical(dv, 7)
            c = lax.bitwise_and(dv, 127)
            plsc.addupdate_scatter(cntloc, [r, c], ones16)

    load_issue(0, 0)
    load_wait(0)
    scatter_issue(0)
    hist(0)
    load_issue(1, 1)
    load_wait(1)
    scatter_issue(1)
    hist(1)
    scatter_wait(0)
    load_issue(2, 0)

    def step(k, carry):
        c = 2 * k
        load_wait(0)
        scatter_issue(0)
        hist(0)
        scatter_wait(1)
        load_issue(c + 1, 1)
        load_wait(1)
        scatter_issue(1)
        hist(1)
        scatter_wait(0)
        load_issue(c + 2, 0)
        return carry

    lax.fori_loop(1, (NCH - 1) // 2, step, 0)
    load_wait(0)
    scatter_issue(0)
    hist(0)
    scatter_wait(1)
    scatter_wait(0)
    pltpu.sync_copy(cntloc, cntsh.at[iota_v], add=True)
    plsc.subcore_barrier()
    for k in range(RPT // RZ):
        r0 = sid * RPT + k * RZ
        pltpu.sync_copy(acc.at[pl.ds(r0, RZ)], zbuf)
        pltpu.sync_copy(zbuf, sums.at[cid, pl.ds(r0, RZ)])

    @pl.when(sid == NS - 1)
    def _():
        tail = N - NS * RPT
        pltpu.sync_copy(acc.at[pl.ds(NS * RPT, tail)], zbuf.at[pl.ds(0, tail)])
        pltpu.sync_copy(zbuf.at[pl.ds(0, tail)],
                        sums.at[cid, pl.ds(NS * RPT, tail)])

    @pl.when(sid == 0)
    def _():
        pltpu.sync_copy(cntsh, cntloc)
        pltpu.sync_copy(cntloc, cnt.at[cid])


@functools.cache
def _scatter():
    return pl.kernel(
        _scatter_body,
        out_type=(jax.ShapeDtypeStruct((NC, N, OUT), jnp.float32),
                  jax.ShapeDtypeStruct((NC, CR, 128), jnp.float32)),
        mesh=_mesh(),
        scratch_types=[pltpu.VMEM((CH,), jnp.int32),
                       pltpu.VMEM((CH, OUT), jnp.float32),
                       pltpu.VMEM((CH,), jnp.int32),
                       pltpu.VMEM((CH, OUT), jnp.float32),
                       pltpu.VMEM((RZ, OUT), jnp.float32),
                       pltpu.VMEM((CR,), jnp.int32),
                       pltpu.VMEM((CR, 128), jnp.float32)]
                      + [pltpu.SemaphoreType.DMA] * 6
                      + [pltpu.VMEM_SHARED((N, OUT), jnp.float32),
                         pltpu.VMEM_SHARED((CR, 128), jnp.float32)],
        compiler_params=pltpu.CompilerParams(needs_layout_passes=False),
    )


_NF = 1000


def _final_body(sums_ref, cnt_ref, out_ref):
    a = sums_ref[...]
    s = a[0] + a[1]
    c = cnt_ref[...]
    c2 = c[0] + c[1]
    out_ref[...] = s / jnp.maximum(c2, 1.0)


def _finalize(sums, cnt_col):
    return pl.pallas_call(
        _final_body,
        grid=(N // _NF,),
        in_specs=[pl.BlockSpec((NC, _NF, OUT), lambda i: (0, i, 0)),
                  pl.BlockSpec((NC, _NF, 1), lambda i: (0, i, 0))],
        out_specs=pl.BlockSpec((_NF, OUT), lambda i: (i, 0)),
        out_shape=jax.ShapeDtypeStruct((N, OUT), jnp.float32),
    )(sums, cnt_col)


def kernel(x, edge_index, edge_attr, W1, b1, W2, b2, W3, b3, W4, b4):
    ei = edge_index.astype(jnp.int32)
    src = ei[0]
    dst = ei[1]
    w1abt = W1[:, :2 * D].T
    w1abt = jnp.concatenate([w1abt[:D], w1abt[D:]], axis=1)
    w1ct = W1[:, 2 * D:].T
    tab = _proj(x, w1abt)
    inter = jnp.stack([src * 2, dst * 2 + 1], axis=1).reshape(2 * E)
    s_arr = _gather()(tab.reshape(2 * N, H), inter).reshape(E, D)
    msg = _edge_mlp(s_arr, edge_attr, w1ct, b1.reshape(1, H),
                    W2.T, b2.reshape(1, H), W3.T, b3.reshape(1, H),
                    W4.T, b4.reshape(1, OUT))
    zrows = jnp.zeros((RZ, OUT), jnp.float32)
    sums, cnt = _scatter()(msg, dst, zrows)
    cnt_col = cnt.reshape(NC, CR * 128)[:, :N].reshape(NC, N, 1)
    return _finalize(sums, cnt_col)
